# baseline (speedup 1.0000x reference)
import functools

import jax
import jax.numpy as jnp
from jax import lax
from jax.experimental import pallas as pl
from jax.experimental.pallas import tpu as pltpu
from jax.experimental.pallas import tpu_sc as plsc

NC = 2
NS = 16
NW = NC * NS
C = 128
GRP = 16


def _fill(ref, n, value):
    def body(i, _):
        ref[pl.ds(i * 16, 16)] = jnp.full((16,), value, jnp.float32)
        return 0
    lax.fori_loop(0, n // 16, body, 0)


def _fill2d(ref, rows, cols, value):
    def body(i, _):
        r = i // (cols // 16)
        c = i % (cols // 16)
        ref[r, pl.ds(c * 16, 16)] = jnp.full((16,), value, jnp.float32)
        return 0
    lax.fori_loop(0, rows * (cols // 16), body, 0)


def _chunk_range(wid, e_chunks):
    base, rem = e_chunks // NW, e_chunks % NW
    start = base * wid + jnp.minimum(wid, rem)
    count = base + jnp.where(wid < rem, 1, 0)
    return start, count


def _sc_degree(edge, np_rows):
    e_chunks = edge.shape[1] // C
    rows_per_tile = np_rows // NS
    mesh = plsc.VectorSubcoreMesh(core_axis_name="c", subcore_axis_name="s")

    @functools.partial(
        pl.kernel,
        out_type=jax.ShapeDtypeStruct((NC, np_rows), jnp.float32),
        mesh=mesh,
        scratch_types=[
            pltpu.VMEM_SHARED((np_rows,), jnp.float32),
            pltpu.VMEM((2, 2, C), jnp.int32),
            pltpu.VMEM((C,), jnp.float32),
            pltpu.VMEM((rows_per_tile,), jnp.float32),
            pltpu.SemaphoreType.DMA,
        ],
    )
    def deg_kernel(edge_hbm, out_hbm, acc, idx_v, ones_v, zeros_v, sem_i):
        cid = lax.axis_index("c")
        sid = lax.axis_index("s")
        wid = cid * NS + sid
        start, count = _chunk_range(wid, e_chunks)
        _fill(ones_v, C, 1.0)
        _fill(zeros_v, rows_per_tile, 0.0)
        pltpu.sync_copy(zeros_v, acc.at[pl.ds(sid * rows_per_tile, rows_per_tile)])
        plsc.subcore_barrier()

        pltpu.async_copy(edge_hbm.at[:, pl.ds(start * C, C)], idx_v.at[0],
                         sem_i)

        def chunk(j, _):
            s = j % 2
            pltpu.make_async_copy(edge_hbm.at[:, pl.ds(0, C)], idx_v.at[s],
                                  sem_i).wait()

            @pl.when(j + 1 < count)
            def _():
                pltpu.async_copy(
                    edge_hbm.at[:, pl.ds((start + j + 1) * C, C)],
                    idx_v.at[1 - s], sem_i)
            pltpu.sync_copy(ones_v, acc.at[idx_v.at[s, 1]], add=True)
            return 0
        lax.fori_loop(0, count, chunk, 0)
        plsc.subcore_barrier()
        pltpu.sync_copy(acc.at[pl.ds(sid * rows_per_tile, rows_per_tile)],
                        out_hbm.at[cid, pl.ds(sid * rows_per_tile, rows_per_tile)])

    return deg_kernel(edge)


def _sc_aggregate(g, edge, np_rows, d):
    e_chunks = edge.shape[1] // C
    rows_per_tile = np_rows // NS
    mesh = plsc.VectorSubcoreMesh(core_axis_name="c", subcore_axis_name="s")

    @functools.partial(
        pl.kernel,
        out_type=jax.ShapeDtypeStruct((NC, np_rows, d), jnp.float32),
        mesh=mesh,
        scratch_types=[
            pltpu.VMEM_SHARED((np_rows, d), jnp.float32),
            pltpu.VMEM((3, 2, C), jnp.int32),
            pltpu.VMEM((2, C, d), jnp.float32),
            pltpu.SemaphoreType.DMA,
            pltpu.SemaphoreType.DMA,
        ],
    )
    def agg_kernel(g_hbm, edge_hbm, out_hbm, acc, idx_v, rows_v, sem_g, sem_i):
        cid = lax.axis_index("c")
        sid = lax.axis_index("s")
        wid = cid * NS + sid
        start, count = _chunk_range(wid, e_chunks)

        _fill2d(rows_v.at[0], C, d, 0.0)
        zfull = rows_per_tile // C
        for k in range(zfull):
            pltpu.sync_copy(rows_v.at[0],
                            acc.at[pl.ds(sid * rows_per_tile + k * C, C)])
        ztail = rows_per_tile - zfull * C
        if ztail:
            pltpu.sync_copy(
                rows_v.at[0].at[pl.ds(0, ztail)],
                acc.at[pl.ds(sid * rows_per_tile + zfull * C, ztail)])
        plsc.subcore_barrier()

        def stage_idx(j, slot):
            pltpu.async_copy(edge_hbm.at[:, pl.ds((start + j) * C, C)],
                             idx_v.at[slot], sem_i)

        def wait_idx(slot):
            pltpu.make_async_copy(edge_hbm.at[:, pl.ds(0, C)],
                                  idx_v.at[slot], sem_i).wait()

        def gather(j, buf):
            pltpu.async_copy(g_hbm.at[idx_v.at[j % 3, 0]], rows_v.at[buf],
                             sem_g)

        def wait_gather(buf):
            pltpu.make_async_copy(g_hbm.at[pl.ds(0, C)], rows_v.at[buf],
                                  sem_g).wait()

        def scatter(j):
            pltpu.sync_copy(rows_v.at[j % 2], acc.at[idx_v.at[j % 3, 1]],
                            add=True)

        stage_idx(0, 0)
        wait_idx(0)
        gather(0, 0)

        @pl.when(count > 1)
        def _():
            stage_idx(1, 1)

        def body(j, _):
            wait_idx((j + 1) % 3)
            wait_gather(j % 2)
            gather(j + 1, (j + 1) % 2)

            @pl.when(j + 2 < count)
            def _():
                stage_idx(j + 2, (j + 2) % 3)
            scatter(j)
            return 0
        lax.fori_loop(0, count - 1, body, 0)
        wait_gather((count - 1) % 2)
        scatter(count - 1)

        plsc.subcore_barrier()
        pltpu.sync_copy(acc.at[pl.ds(sid * rows_per_tile, rows_per_tile)],
                        out_hbm.at[cid, pl.ds(sid * rows_per_tile, rows_per_tile)])

    return agg_kernel(g, edge)


def _tc_first(degsum, x, w0, n, blk):
    din, dh = w0.shape

    def body(deg_ref, x_ref, w_ref, dinv_ref, g_ref):
        dv = lax.rsqrt(deg_ref[...] + 1.0)
        dinv_ref[...] = dv
        h = jnp.dot(x_ref[...], w_ref[...], preferred_element_type=jnp.float32)
        g_ref[...] = h * dv

    grid = (n // blk,)
    return pl.pallas_call(
        body,
        grid=grid,
        in_specs=[
            pl.BlockSpec((blk, 1), lambda i: (i, 0)),
            pl.BlockSpec((blk, din), lambda i: (i, 0)),
            pl.BlockSpec((din, dh), lambda i: (0, 0)),
        ],
        out_specs=[
            pl.BlockSpec((blk, 1), lambda i: (i, 0)),
            pl.BlockSpec((blk, dh), lambda i: (i, 0)),
        ],
        out_shape=[
            jax.ShapeDtypeStruct((n, 1), jnp.float32),
            jax.ShapeDtypeStruct((n, dh), jnp.float32),
        ],
    )(degsum, x, w0)


def _tc_mid(aggp, g, dinv, b, w, n, blk):
    d, dn = w.shape

    def body(aggp_ref, g_ref, dinv_ref, b_ref, w_ref, out_ref):
        s = aggp_ref[0] + aggp_ref[1] + g_ref[...]
        dv = dinv_ref[...]
        h = jnp.maximum(s * dv + b_ref[...][None, :], 0.0)
        out_ref[...] = jnp.dot(h, w_ref[...],
                               preferred_element_type=jnp.float32) * dv

    grid = (n // blk,)
    return pl.pallas_call(
        body,
        grid=grid,
        in_specs=[
            pl.BlockSpec((NC, blk, d), lambda i: (0, i, 0)),
            pl.BlockSpec((blk, d), lambda i: (i, 0)),
            pl.BlockSpec((blk, 1), lambda i: (i, 0)),
            pl.BlockSpec((d,), lambda i: (0,)),
            pl.BlockSpec((d, dn), lambda i: (0, 0)),
        ],
        out_specs=pl.BlockSpec((blk, dn), lambda i: (i, 0)),
        out_shape=jax.ShapeDtypeStruct((n, dn), jnp.float32),
    )(aggp, g, dinv, b, w)


def _tc_final(aggp, g, dinv, b, n, blk):
    d = g.shape[1]
    dout = b.shape[0]

    def body(aggp_ref, g_ref, dinv_ref, b_ref, out_ref):
        s = aggp_ref[0] + aggp_ref[1] + g_ref[...]
        v = (s * dinv_ref[...])[:, :dout] + b_ref[...][None, :]
        m = jnp.max(v, axis=-1, keepdims=True)
        e = v - m
        out_ref[...] = e - jnp.log(jnp.sum(jnp.exp(e), axis=-1, keepdims=True))

    grid = (n // blk,)
    return pl.pallas_call(
        body,
        grid=grid,
        in_specs=[
            pl.BlockSpec((NC, blk, d), lambda i: (0, i, 0)),
            pl.BlockSpec((blk, d), lambda i: (i, 0)),
            pl.BlockSpec((blk, 1), lambda i: (i, 0)),
            pl.BlockSpec((dout,), lambda i: (0,)),
        ],
        out_specs=pl.BlockSpec((blk, dout), lambda i: (i, 0)),
        out_shape=jax.ShapeDtypeStruct((n, dout), jnp.float32),
    )(aggp, g, dinv, b)


def kernel(x, edge_index, W0, b0, W1, b1, W2, b2):
    n, din = x.shape

    np_rows = (n + 16 * NW - 1) // (16 * NW) * (16 * NW)

    blk = 1000
    degp = _sc_degree(edge_index, np_rows)
    degsum = (degp[0] + degp[1]).reshape(np_rows, 1)
    dinv, g0 = _tc_first(degsum, x, W0, n, blk)
    a0 = _sc_aggregate(g0, edge_index, np_rows, W0.shape[1])
    g1 = _tc_mid(a0, g0, dinv, b0, W1, n, blk)
    a1 = _sc_aggregate(g1, edge_index, np_rows, W1.shape[1])
    W2p = jnp.pad(W2, ((0, 0), (0, 128 - W2.shape[1])))
    g2 = _tc_mid(a1, g1, dinv, b1, W2p, n, blk)
    a2 = _sc_aggregate(g2, edge_index, np_rows, W2p.shape[1])
    return _tc_final(a2, g2, dinv, b2, n, blk)


# speedup vs baseline: 1.1457x; 1.1457x over previous
import functools

import jax
import jax.numpy as jnp
from jax import lax
from jax.experimental import pallas as pl
from jax.experimental.pallas import tpu as pltpu
from jax.experimental.pallas import tpu_sc as plsc

NC = 2
NS = 16
NW = NC * NS
C = 128
GRP = 16


def _fill(ref, n, value):
    def body(i, _):
        ref[pl.ds(i * 16, 16)] = jnp.full((16,), value, jnp.float32)
        return 0
    lax.fori_loop(0, n // 16, body, 0)


def _fill2d(ref, rows, cols, value):
    def body(i, _):
        r = i // (cols // 16)
        c = i % (cols // 16)
        ref[r, pl.ds(c * 16, 16)] = jnp.full((16,), value, jnp.float32)
        return 0
    lax.fori_loop(0, rows * (cols // 16), body, 0)


def _sc_degree(edge3, np_rows, nch):
    rows_per_tile = np_rows // NS
    mesh = plsc.VectorSubcoreMesh(core_axis_name="c", subcore_axis_name="s")

    @functools.partial(
        pl.kernel,
        out_type=jax.ShapeDtypeStruct((NC, np_rows), jnp.float32),
        mesh=mesh,
        scratch_types=[
            pltpu.VMEM_SHARED((np_rows,), jnp.float32),
            pltpu.VMEM((GRP, 2, C), jnp.int32),
            pltpu.VMEM((C,), jnp.float32),
            pltpu.VMEM((rows_per_tile,), jnp.float32),
        ],
    )
    def deg_kernel(edge_hbm, out_hbm, acc, idx_v, ones_v, zeros_v):
        cid = lax.axis_index("c")
        sid = lax.axis_index("s")
        wid = cid * NS + sid
        _fill(ones_v, C, 1.0)
        _fill(zeros_v, rows_per_tile, 0.0)
        pltpu.sync_copy(zeros_v, acc.at[pl.ds(sid * rows_per_tile, rows_per_tile)])
        plsc.subcore_barrier()

        def group(gi, _):
            pltpu.sync_copy(
                edge_hbm.at[pl.ds(wid * nch + gi * GRP, GRP)], idx_v)

            def chunk(j, _):
                pltpu.sync_copy(ones_v, acc.at[idx_v.at[j, 1]], add=True)
                return 0
            lax.fori_loop(0, GRP, chunk, 0)
            return 0
        lax.fori_loop(0, nch // GRP, group, 0)
        plsc.subcore_barrier()
        pltpu.sync_copy(acc.at[pl.ds(sid * rows_per_tile, rows_per_tile)],
                        out_hbm.at[cid, pl.ds(sid * rows_per_tile, rows_per_tile)])

    return deg_kernel(edge3)


def _sc_aggregate(g, edge3, np_rows, nch, d):
    rows_per_tile = np_rows // NS
    mesh = plsc.VectorSubcoreMesh(core_axis_name="c", subcore_axis_name="s")

    @functools.partial(
        pl.kernel,
        out_type=jax.ShapeDtypeStruct((NC, np_rows, d), jnp.float32),
        mesh=mesh,
        scratch_types=[
            pltpu.VMEM_SHARED((np_rows, d), jnp.float32),
            pltpu.VMEM((2, GRP, 2, C), jnp.int32),
            pltpu.VMEM((2, C, d), jnp.float32),
            pltpu.SemaphoreType.DMA,
            pltpu.SemaphoreType.DMA,
            pltpu.SemaphoreType.DMA,
        ],
    )
    def agg_kernel(g_hbm, edge_hbm, out_hbm, acc, idx_v, rows_v,
                   sem0, sem1, sem_i):
        cid = lax.axis_index("c")
        sid = lax.axis_index("s")
        wid = cid * NS + sid

        _fill2d(rows_v.at[0], C, d, 0.0)
        for k in range(rows_per_tile // C):
            pltpu.sync_copy(rows_v.at[0],
                            acc.at[pl.ds(sid * rows_per_tile + k * C, C)])
        plsc.subcore_barrier()

        def wait_gather(buf, sem):
            pltpu.make_async_copy(g_hbm.at[pl.ds(0, C)], buf, sem).wait()

        ngroups = nch // GRP
        pltpu.sync_copy(edge_hbm.at[pl.ds(wid * nch, GRP)], idx_v.at[0])

        def group(gi, _):
            s = gi % 2
            iv = idx_v.at[s]

            @pl.when(gi > 0)
            def _():
                pltpu.make_async_copy(edge_hbm.at[pl.ds(0, GRP)], iv,
                                      sem_i).wait()

            @pl.when(gi + 1 < ngroups)
            def _():
                pltpu.async_copy(
                    edge_hbm.at[pl.ds(wid * nch + (gi + 1) * GRP, GRP)],
                    idx_v.at[1 - s], sem_i)

            pltpu.async_copy(g_hbm.at[iv.at[0, 0]], rows_v.at[0], sem0)

            def pair(t, _):
                pltpu.async_copy(g_hbm.at[iv.at[2 * t + 1, 0]], rows_v.at[1],
                                 sem1)
                wait_gather(rows_v.at[0], sem0)
                pltpu.sync_copy(rows_v.at[0], acc.at[iv.at[2 * t, 1]],
                                add=True)

                @pl.when(t + 1 < GRP // 2)
                def _():
                    pltpu.async_copy(g_hbm.at[iv.at[2 * t + 2, 0]],
                                     rows_v.at[0], sem0)
                wait_gather(rows_v.at[1], sem1)
                pltpu.sync_copy(rows_v.at[1], acc.at[iv.at[2 * t + 1, 1]],
                                add=True)
                return 0
            lax.fori_loop(0, GRP // 2, pair, 0)
            return 0
        lax.fori_loop(0, ngroups, group, 0)
        plsc.subcore_barrier()
        pltpu.sync_copy(acc.at[pl.ds(sid * rows_per_tile, rows_per_tile)],
                        out_hbm.at[cid, pl.ds(sid * rows_per_tile, rows_per_tile)])

    return agg_kernel(g, edge3)


def _tc_first(degsum, x, w0, n, blk):
    din, dh = w0.shape

    def body(deg_ref, x_ref, w_ref, dinv_ref, g_ref):
        dv = lax.rsqrt(deg_ref[...] + 1.0)
        dinv_ref[...] = dv
        h = jnp.dot(x_ref[...], w_ref[...], preferred_element_type=jnp.float32)
        g_ref[...] = h * dv

    grid = (n // blk,)
    return pl.pallas_call(
        body,
        grid=grid,
        in_specs=[
            pl.BlockSpec((blk, 1), lambda i: (i, 0)),
            pl.BlockSpec((blk, din), lambda i: (i, 0)),
            pl.BlockSpec((din, dh), lambda i: (0, 0)),
        ],
        out_specs=[
            pl.BlockSpec((blk, 1), lambda i: (i, 0)),
            pl.BlockSpec((blk, dh), lambda i: (i, 0)),
        ],
        out_shape=[
            jax.ShapeDtypeStruct((n, 1), jnp.float32),
            jax.ShapeDtypeStruct((n, dh), jnp.float32),
        ],
    )(degsum, x, w0)


def _tc_mid(aggp, g, dinv, b, w, n, blk):
    d, dn = w.shape

    def body(aggp_ref, g_ref, dinv_ref, b_ref, w_ref, out_ref):
        s = aggp_ref[0] + aggp_ref[1] + g_ref[...]
        dv = dinv_ref[...]
        h = jnp.maximum(s * dv + b_ref[...][None, :], 0.0)
        out_ref[...] = jnp.dot(h, w_ref[...],
                               preferred_element_type=jnp.float32) * dv

    grid = (n // blk,)
    return pl.pallas_call(
        body,
        grid=grid,
        in_specs=[
            pl.BlockSpec((NC, blk, d), lambda i: (0, i, 0)),
            pl.BlockSpec((blk, d), lambda i: (i, 0)),
            pl.BlockSpec((blk, 1), lambda i: (i, 0)),
            pl.BlockSpec((d,), lambda i: (0,)),
            pl.BlockSpec((d, dn), lambda i: (0, 0)),
        ],
        out_specs=pl.BlockSpec((blk, dn), lambda i: (i, 0)),
        out_shape=jax.ShapeDtypeStruct((n, dn), jnp.float32),
    )(aggp, g, dinv, b, w)


def _tc_final(aggp, g, dinv, b, n, blk):
    d = g.shape[1]
    dout = b.shape[0]

    def body(aggp_ref, g_ref, dinv_ref, b_ref, out_ref):
        s = aggp_ref[0] + aggp_ref[1] + g_ref[...]
        v = (s * dinv_ref[...])[:, :dout] + b_ref[...][None, :]
        m = jnp.max(v, axis=-1, keepdims=True)
        e = v - m
        out_ref[...] = e - jnp.log(jnp.sum(jnp.exp(e), axis=-1, keepdims=True))

    grid = (n // blk,)
    return pl.pallas_call(
        body,
        grid=grid,
        in_specs=[
            pl.BlockSpec((NC, blk, d), lambda i: (0, i, 0)),
            pl.BlockSpec((blk, d), lambda i: (i, 0)),
            pl.BlockSpec((blk, 1), lambda i: (i, 0)),
            pl.BlockSpec((dout,), lambda i: (0,)),
        ],
        out_specs=pl.BlockSpec((blk, dout), lambda i: (i, 0)),
        out_shape=jax.ShapeDtypeStruct((n, dout), jnp.float32),
    )(aggp, g, dinv, b)


def kernel(x, edge_index, W0, b0, W1, b1, W2, b2):
    n, din = x.shape
    e = edge_index.shape[1]

    np_rows = ((n + 16) + 16 * NW - 1) // (16 * NW) * (16 * NW)
    n_trash = np_rows - n

    e_chunks = e // C
    nch = (e_chunks + NW * GRP - 1) // (NW * GRP) * GRP
    padc = NW * nch - e_chunks
    er = edge_index.reshape(2, e_chunks, C).transpose(1, 0, 2)
    ar = jnp.arange(padc * C, dtype=jnp.int32)
    pad3 = jnp.stack([(ar % n).reshape(padc, C),
                      (n + ar % n_trash).reshape(padc, C)], axis=1)
    edge3 = jnp.concatenate([er, pad3], axis=0)

    blk = 1000
    degp = _sc_degree(edge3, np_rows, nch)
    degsum = (degp[0] + degp[1]).reshape(np_rows, 1)
    dinv, g0 = _tc_first(degsum, x, W0, n, blk)
    a0 = _sc_aggregate(g0, edge3, np_rows, nch, W0.shape[1])
    g1 = _tc_mid(a0, g0, dinv, b0, W1, n, blk)
    a1 = _sc_aggregate(g1, edge3, np_rows, nch, W1.shape[1])
    W2p = jnp.pad(W2, ((0, 0), (0, 128 - W2.shape[1])))
    g2 = _tc_mid(a1, g1, dinv, b1, W2p, n, blk)
    a2 = _sc_aggregate(g2, edge3, np_rows, nch, W2p.shape[1])
    return _tc_final(a2, g2, dinv, b2, n, blk)


# speedup vs baseline: 1.1467x; 1.0009x over previous
import functools

import jax
import jax.numpy as jnp
from jax import lax
from jax.experimental import pallas as pl
from jax.experimental.pallas import tpu as pltpu
from jax.experimental.pallas import tpu_sc as plsc

NC = 2
NS = 16
NW = NC * NS
C = 128
GRP = 16


def _fill(ref, n, value):
    def body(i, _):
        ref[pl.ds(i * 16, 16)] = jnp.full((16,), value, jnp.float32)
        return 0
    lax.fori_loop(0, n // 16, body, 0)


def _fill2d(ref, rows, cols, value):
    def body(i, _):
        r = i // (cols // 16)
        c = i % (cols // 16)
        ref[r, pl.ds(c * 16, 16)] = jnp.full((16,), value, jnp.float32)
        return 0
    lax.fori_loop(0, rows * (cols // 16), body, 0)


def _sc_degree(edge3, np_rows, nch):
    rows_per_tile = np_rows // NS
    mesh = plsc.VectorSubcoreMesh(core_axis_name="c", subcore_axis_name="s")

    @functools.partial(
        pl.kernel,
        out_type=jax.ShapeDtypeStruct((NC, np_rows), jnp.float32),
        mesh=mesh,
        scratch_types=[
            pltpu.VMEM_SHARED((np_rows,), jnp.float32),
            pltpu.VMEM((GRP, 2, C), jnp.int32),
            pltpu.VMEM((C,), jnp.float32),
            pltpu.VMEM((rows_per_tile,), jnp.float32),
        ],
    )
    def deg_kernel(edge_hbm, out_hbm, acc, idx_v, ones_v, zeros_v):
        cid = lax.axis_index("c")
        sid = lax.axis_index("s")
        wid = cid * NS + sid
        _fill(ones_v, C, 1.0)
        _fill(zeros_v, rows_per_tile, 0.0)
        pltpu.sync_copy(zeros_v, acc.at[pl.ds(sid * rows_per_tile, rows_per_tile)])
        plsc.subcore_barrier()

        def group(gi, _):
            pltpu.sync_copy(
                edge_hbm.at[pl.ds(wid * nch + gi * GRP, GRP)], idx_v)

            def chunk(j, _):
                pltpu.sync_copy(ones_v, acc.at[idx_v.at[j, 1]], add=True)
                return 0
            lax.fori_loop(0, GRP, chunk, 0)
            return 0
        lax.fori_loop(0, nch // GRP, group, 0)
        plsc.subcore_barrier()
        pltpu.sync_copy(acc.at[pl.ds(sid * rows_per_tile, rows_per_tile)],
                        out_hbm.at[cid, pl.ds(sid * rows_per_tile, rows_per_tile)])

    return deg_kernel(edge3)


def _sc_aggregate(g, edge3, np_rows, nch, d):
    rows_per_tile = np_rows // NS
    mesh = plsc.VectorSubcoreMesh(core_axis_name="c", subcore_axis_name="s")

    @functools.partial(
        pl.kernel,
        out_type=jax.ShapeDtypeStruct((NC, np_rows, d), jnp.float32),
        mesh=mesh,
        scratch_types=[
            pltpu.VMEM_SHARED((np_rows, d), jnp.float32),
            pltpu.VMEM((2, GRP, 2, C), jnp.int32),
            pltpu.VMEM((2, C, d), jnp.float32),
            pltpu.SemaphoreType.DMA,
            pltpu.SemaphoreType.DMA,
            pltpu.SemaphoreType.DMA,
        ],
    )
    def agg_kernel(g_hbm, edge_hbm, out_hbm, acc, idx_v, rows_v,
                   sem0, sem1, sem_i):
        cid = lax.axis_index("c")
        sid = lax.axis_index("s")
        wid = cid * NS + sid

        _fill2d(rows_v.at[0], C, d, 0.0)
        for k in range(rows_per_tile // C):
            pltpu.sync_copy(rows_v.at[0],
                            acc.at[pl.ds(sid * rows_per_tile + k * C, C)])
        plsc.subcore_barrier()

        def wait_gather(buf, sem):
            pltpu.make_async_copy(g_hbm.at[pl.ds(0, C)], buf, sem).wait()

        ngroups = nch // GRP
        pltpu.sync_copy(edge_hbm.at[pl.ds(wid * nch, GRP)], idx_v.at[0])

        def group(gi, _):
            s = gi % 2
            iv = idx_v.at[s]

            @pl.when(gi > 0)
            def _():
                pltpu.make_async_copy(edge_hbm.at[pl.ds(0, GRP)], iv,
                                      sem_i).wait()

            @pl.when(gi + 1 < ngroups)
            def _():
                pltpu.async_copy(
                    edge_hbm.at[pl.ds(wid * nch + (gi + 1) * GRP, GRP)],
                    idx_v.at[1 - s], sem_i)

            pltpu.async_copy(g_hbm.at[iv.at[0, 0]], rows_v.at[0], sem0)

            def pair(t, _):
                pltpu.async_copy(g_hbm.at[iv.at[2 * t + 1, 0]], rows_v.at[1],
                                 sem1)
                wait_gather(rows_v.at[0], sem0)
                pltpu.sync_copy(rows_v.at[0], acc.at[iv.at[2 * t, 1]],
                                add=True)

                @pl.when(t + 1 < GRP // 2)
                def _():
                    pltpu.async_copy(g_hbm.at[iv.at[2 * t + 2, 0]],
                                     rows_v.at[0], sem0)
                wait_gather(rows_v.at[1], sem1)
                pltpu.sync_copy(rows_v.at[1], acc.at[iv.at[2 * t + 1, 1]],
                                add=True)
                return 0
            lax.fori_loop(0, GRP // 2, pair, 0)
            return 0
        lax.fori_loop(0, ngroups, group, 0)
        plsc.subcore_barrier()
        pltpu.sync_copy(acc.at[pl.ds(sid * rows_per_tile, rows_per_tile)],
                        out_hbm.at[cid, pl.ds(sid * rows_per_tile, rows_per_tile)])

    return agg_kernel(g, edge3)


def _tc_matmul(x, w0, n, blk):
    din, dh = w0.shape

    def body(x_ref, w_ref, h_ref):
        h_ref[...] = jnp.dot(x_ref[...], w_ref[...],
                             preferred_element_type=jnp.float32)

    grid = (n // blk,)
    return pl.pallas_call(
        body,
        grid=grid,
        in_specs=[
            pl.BlockSpec((blk, din), lambda i: (i, 0)),
            pl.BlockSpec((din, dh), lambda i: (0, 0)),
        ],
        out_specs=pl.BlockSpec((blk, dh), lambda i: (i, 0)),
        out_shape=jax.ShapeDtypeStruct((n, dh), jnp.float32),
    )(x, w0)


def _tc_scale(degsum, h0, n, blk):
    dh = h0.shape[1]

    def body(deg_ref, h_ref, dinv_ref, g_ref):
        dv = lax.rsqrt(deg_ref[...] + 1.0)
        dinv_ref[...] = dv
        g_ref[...] = h_ref[...] * dv

    grid = (n // blk,)
    return pl.pallas_call(
        body,
        grid=grid,
        in_specs=[
            pl.BlockSpec((blk, 1), lambda i: (i, 0)),
            pl.BlockSpec((blk, dh), lambda i: (i, 0)),
        ],
        out_specs=[
            pl.BlockSpec((blk, 1), lambda i: (i, 0)),
            pl.BlockSpec((blk, dh), lambda i: (i, 0)),
        ],
        out_shape=[
            jax.ShapeDtypeStruct((n, 1), jnp.float32),
            jax.ShapeDtypeStruct((n, dh), jnp.float32),
        ],
    )(degsum, h0)


def _tc_mid(aggp, g, dinv, b, w, n, blk):
    d, dn = w.shape

    def body(aggp_ref, g_ref, dinv_ref, b_ref, w_ref, out_ref):
        s = aggp_ref[0] + aggp_ref[1] + g_ref[...]
        dv = dinv_ref[...]
        h = jnp.maximum(s * dv + b_ref[...][None, :], 0.0)
        out_ref[...] = jnp.dot(h, w_ref[...],
                               preferred_element_type=jnp.float32) * dv

    grid = (n // blk,)
    return pl.pallas_call(
        body,
        grid=grid,
        in_specs=[
            pl.BlockSpec((NC, blk, d), lambda i: (0, i, 0)),
            pl.BlockSpec((blk, d), lambda i: (i, 0)),
            pl.BlockSpec((blk, 1), lambda i: (i, 0)),
            pl.BlockSpec((d,), lambda i: (0,)),
            pl.BlockSpec((d, dn), lambda i: (0, 0)),
        ],
        out_specs=pl.BlockSpec((blk, dn), lambda i: (i, 0)),
        out_shape=jax.ShapeDtypeStruct((n, dn), jnp.float32),
    )(aggp, g, dinv, b, w)


def _tc_final(aggp, g, dinv, b, n, blk):
    d = g.shape[1]
    dout = b.shape[0]

    def body(aggp_ref, g_ref, dinv_ref, b_ref, out_ref):
        s = aggp_ref[0] + aggp_ref[1] + g_ref[...]
        v = (s * dinv_ref[...])[:, :dout] + b_ref[...][None, :]
        m = jnp.max(v, axis=-1, keepdims=True)
        e = v - m
        out_ref[...] = e - jnp.log(jnp.sum(jnp.exp(e), axis=-1, keepdims=True))

    grid = (n // blk,)
    return pl.pallas_call(
        body,
        grid=grid,
        in_specs=[
            pl.BlockSpec((NC, blk, d), lambda i: (0, i, 0)),
            pl.BlockSpec((blk, d), lambda i: (i, 0)),
            pl.BlockSpec((blk, 1), lambda i: (i, 0)),
            pl.BlockSpec((dout,), lambda i: (0,)),
        ],
        out_specs=pl.BlockSpec((blk, dout), lambda i: (i, 0)),
        out_shape=jax.ShapeDtypeStruct((n, dout), jnp.float32),
    )(aggp, g, dinv, b)


def kernel(x, edge_index, W0, b0, W1, b1, W2, b2):
    n, din = x.shape
    e = edge_index.shape[1]

    np_rows = ((n + 16) + 16 * NW - 1) // (16 * NW) * (16 * NW)
    n_trash = np_rows - n

    e_chunks = e // C
    nch = (e_chunks + NW * GRP - 1) // (NW * GRP) * GRP
    padc = NW * nch - e_chunks
    er = edge_index.reshape(2, e_chunks, C).transpose(1, 0, 2)
    ar = jnp.arange(padc * C, dtype=jnp.int32)
    pad3 = jnp.stack([(ar % n).reshape(padc, C),
                      (n + ar % n_trash).reshape(padc, C)], axis=1)
    edge3 = jnp.concatenate([er, pad3], axis=0)

    blk = 1000
    degp = _sc_degree(edge3, np_rows, nch)
    h0 = _tc_matmul(x, W0, n, blk)
    degsum = (degp[0] + degp[1]).reshape(np_rows, 1)
    dinv, g0 = _tc_scale(degsum, h0, n, blk)
    a0 = _sc_aggregate(g0, edge3, np_rows, nch, W0.shape[1])
    g1 = _tc_mid(a0, g0, dinv, b0, W1, n, blk)
    a1 = _sc_aggregate(g1, edge3, np_rows, nch, W1.shape[1])
    W2p = jnp.pad(W2, ((0, 0), (0, 128 - W2.shape[1])))
    g2 = _tc_mid(a1, g1, dinv, b1, W2p, n, blk)
    a2 = _sc_aggregate(g2, edge3, np_rows, nch, W2p.shape[1])
    return _tc_final(a2, g2, dinv, b2, n, blk)


# speedup vs baseline: 1.1583x; 1.0101x over previous
import functools

import jax
import jax.numpy as jnp
from jax import lax
from jax.experimental import pallas as pl
from jax.experimental.pallas import tpu as pltpu
from jax.experimental.pallas import tpu_sc as plsc

NC = 2
NS = 16
NW = NC * NS
C = 128
GRP = 16


def _fill(ref, n, value):
    def body(i, _):
        ref[pl.ds(i * 16, 16)] = jnp.full((16,), value, jnp.float32)
        return 0
    lax.fori_loop(0, n // 16, body, 0)


def _fill2d(ref, rows, cols, value):
    def body(i, _):
        r = i // (cols // 16)
        c = i % (cols // 16)
        ref[r, pl.ds(c * 16, 16)] = jnp.full((16,), value, jnp.float32)
        return 0
    lax.fori_loop(0, rows * (cols // 16), body, 0)


def _sc_degree(edge3, np_rows, base, rem):
    rows_per_tile = np_rows // NS
    full_groups, tail = base // GRP, base % GRP
    mesh = plsc.VectorSubcoreMesh(core_axis_name="c", subcore_axis_name="s")

    @functools.partial(
        pl.kernel,
        out_type=jax.ShapeDtypeStruct((NC, np_rows), jnp.float32),
        mesh=mesh,
        scratch_types=[
            pltpu.VMEM_SHARED((np_rows,), jnp.float32),
            pltpu.VMEM((GRP, 2, C), jnp.int32),
            pltpu.VMEM((C,), jnp.float32),
            pltpu.VMEM((rows_per_tile,), jnp.float32),
        ],
    )
    def deg_kernel(edge_hbm, out_hbm, acc, idx_v, ones_v, zeros_v):
        cid = lax.axis_index("c")
        sid = lax.axis_index("s")
        wid = cid * NS + sid
        start = base * wid + jnp.minimum(wid, rem)
        _fill(ones_v, C, 1.0)
        _fill(zeros_v, rows_per_tile, 0.0)
        pltpu.sync_copy(zeros_v, acc.at[pl.ds(sid * rows_per_tile, rows_per_tile)])
        plsc.subcore_barrier()

        def run(nchunks):
            def chunk(j, _):
                pltpu.sync_copy(ones_v, acc.at[idx_v.at[j, 1]], add=True)
                return 0
            lax.fori_loop(0, nchunks, chunk, 0)

        def group(gi, _):
            pltpu.sync_copy(edge_hbm.at[pl.ds(start + gi * GRP, GRP)], idx_v)
            run(GRP)
            return 0
        lax.fori_loop(0, full_groups, group, 0)
        if tail:
            pltpu.sync_copy(
                edge_hbm.at[pl.ds(start + full_groups * GRP, tail)],
                idx_v.at[pl.ds(0, tail)])
            run(tail)
        if rem:
            @pl.when(wid < rem)
            def _():
                pltpu.sync_copy(edge_hbm.at[pl.ds(start + base, 1)],
                                idx_v.at[pl.ds(0, 1)])
                pltpu.sync_copy(ones_v, acc.at[idx_v.at[0, 1]], add=True)
        plsc.subcore_barrier()
        pltpu.sync_copy(acc.at[pl.ds(sid * rows_per_tile, rows_per_tile)],
                        out_hbm.at[cid, pl.ds(sid * rows_per_tile, rows_per_tile)])

    return deg_kernel(edge3)


def _sc_aggregate(g, edge3, np_rows, base, rem, d):
    rows_per_tile = np_rows // NS
    full_groups, tail = base // GRP, base % GRP
    mesh = plsc.VectorSubcoreMesh(core_axis_name="c", subcore_axis_name="s")

    @functools.partial(
        pl.kernel,
        out_type=jax.ShapeDtypeStruct((NC, np_rows, d), jnp.float32),
        mesh=mesh,
        scratch_types=[
            pltpu.VMEM_SHARED((np_rows, d), jnp.float32),
            pltpu.VMEM((2, GRP, 2, C), jnp.int32),
            pltpu.VMEM((2, C, d), jnp.float32),
            pltpu.SemaphoreType.DMA,
            pltpu.SemaphoreType.DMA,
            pltpu.SemaphoreType.DMA,
        ],
    )
    def agg_kernel(g_hbm, edge_hbm, out_hbm, acc, idx_v, rows_v,
                   sem0, sem1, sem_i):
        cid = lax.axis_index("c")
        sid = lax.axis_index("s")
        wid = cid * NS + sid
        start = base * wid + jnp.minimum(wid, rem)

        _fill2d(rows_v.at[0], C, d, 0.0)
        for k in range(rows_per_tile // C):
            pltpu.sync_copy(rows_v.at[0],
                            acc.at[pl.ds(sid * rows_per_tile + k * C, C)])
        plsc.subcore_barrier()

        def wait_gather(buf, sem):
            pltpu.make_async_copy(g_hbm.at[pl.ds(0, C)], buf, sem).wait()

        def pairs(iv, npairs):
            pltpu.async_copy(g_hbm.at[iv.at[0, 0]], rows_v.at[0], sem0)

            def pair(t, _):
                pltpu.async_copy(g_hbm.at[iv.at[2 * t + 1, 0]], rows_v.at[1],
                                 sem1)
                wait_gather(rows_v.at[0], sem0)
                pltpu.sync_copy(rows_v.at[0], acc.at[iv.at[2 * t, 1]],
                                add=True)

                @pl.when(t + 1 < npairs)
                def _():
                    pltpu.async_copy(g_hbm.at[iv.at[2 * t + 2, 0]],
                                     rows_v.at[0], sem0)
                wait_gather(rows_v.at[1], sem1)
                pltpu.sync_copy(rows_v.at[1], acc.at[iv.at[2 * t + 1, 1]],
                                add=True)
                return 0
            lax.fori_loop(0, npairs, pair, 0)

        pltpu.sync_copy(edge_hbm.at[pl.ds(start, GRP)], idx_v.at[0])

        def group(gi, _):
            s = gi % 2
            iv = idx_v.at[s]

            @pl.when(gi > 0)
            def _():
                pltpu.make_async_copy(edge_hbm.at[pl.ds(0, GRP)], iv,
                                      sem_i).wait()

            @pl.when(gi + 1 < full_groups)
            def _():
                pltpu.async_copy(
                    edge_hbm.at[pl.ds(start + (gi + 1) * GRP, GRP)],
                    idx_v.at[1 - s], sem_i)
            if tail:
                @pl.when(gi + 1 == full_groups)
                def _():
                    pltpu.async_copy(
                        edge_hbm.at[pl.ds(start + full_groups * GRP, tail)],
                        idx_v.at[1 - s].at[pl.ds(0, tail)], sem_i)
            pairs(iv, GRP // 2)
            return 0
        lax.fori_loop(0, full_groups, group, 0)

        if tail:
            s = full_groups % 2
            iv = idx_v.at[s]
            if full_groups:
                pltpu.make_async_copy(edge_hbm.at[pl.ds(0, tail)],
                                      iv.at[pl.ds(0, tail)], sem_i).wait()
            else:
                pltpu.sync_copy(edge_hbm.at[pl.ds(start, tail)],
                                iv.at[pl.ds(0, tail)])
            pairs(iv, tail // 2)
            if tail % 2:
                pltpu.async_copy(g_hbm.at[iv.at[tail - 1, 0]], rows_v.at[0],
                                 sem0)
                wait_gather(rows_v.at[0], sem0)
                pltpu.sync_copy(rows_v.at[0], acc.at[iv.at[tail - 1, 1]],
                                add=True)
        if rem:
            @pl.when(wid < rem)
            def _():
                pltpu.sync_copy(edge_hbm.at[pl.ds(start + base, 1)],
                                idx_v.at[0].at[pl.ds(0, 1)])
                pltpu.async_copy(g_hbm.at[idx_v.at[0, 0, 0]], rows_v.at[0],
                                 sem0)
                wait_gather(rows_v.at[0], sem0)
                pltpu.sync_copy(rows_v.at[0], acc.at[idx_v.at[0, 0, 1]],
                                add=True)
        plsc.subcore_barrier()
        pltpu.sync_copy(acc.at[pl.ds(sid * rows_per_tile, rows_per_tile)],
                        out_hbm.at[cid, pl.ds(sid * rows_per_tile, rows_per_tile)])

    return agg_kernel(g, edge3)


def _tc_matmul(x, w0, n, blk):
    din, dh = w0.shape

    def body(x_ref, w_ref, h_ref):
        h_ref[...] = jnp.dot(x_ref[...], w_ref[...],
                             preferred_element_type=jnp.float32)

    grid = (n // blk,)
    return pl.pallas_call(
        body,
        grid=grid,
        in_specs=[
            pl.BlockSpec((blk, din), lambda i: (i, 0)),
            pl.BlockSpec((din, dh), lambda i: (0, 0)),
        ],
        out_specs=pl.BlockSpec((blk, dh), lambda i: (i, 0)),
        out_shape=jax.ShapeDtypeStruct((n, dh), jnp.float32),
    )(x, w0)


def _tc_scale(degsum, h0, n, blk):
    dh = h0.shape[1]

    def body(deg_ref, h_ref, dinv_ref, g_ref):
        dv = lax.rsqrt(deg_ref[...] + 1.0)
        dinv_ref[...] = dv
        g_ref[...] = h_ref[...] * dv

    grid = (n // blk,)
    return pl.pallas_call(
        body,
        grid=grid,
        in_specs=[
            pl.BlockSpec((blk, 1), lambda i: (i, 0)),
            pl.BlockSpec((blk, dh), lambda i: (i, 0)),
        ],
        out_specs=[
            pl.BlockSpec((blk, 1), lambda i: (i, 0)),
            pl.BlockSpec((blk, dh), lambda i: (i, 0)),
        ],
        out_shape=[
            jax.ShapeDtypeStruct((n, 1), jnp.float32),
            jax.ShapeDtypeStruct((n, dh), jnp.float32),
        ],
    )(degsum, h0)


def _tc_mid(aggp, g, dinv, b, w, n, blk):
    d, dn = w.shape

    def body(aggp_ref, g_ref, dinv_ref, b_ref, w_ref, out_ref):
        s = aggp_ref[0] + aggp_ref[1] + g_ref[...]
        dv = dinv_ref[...]
        h = jnp.maximum(s * dv + b_ref[...][None, :], 0.0)
        out_ref[...] = jnp.dot(h, w_ref[...],
                               preferred_element_type=jnp.float32) * dv

    grid = (n // blk,)
    return pl.pallas_call(
        body,
        grid=grid,
        in_specs=[
            pl.BlockSpec((NC, blk, d), lambda i: (0, i, 0)),
            pl.BlockSpec((blk, d), lambda i: (i, 0)),
            pl.BlockSpec((blk, 1), lambda i: (i, 0)),
            pl.BlockSpec((d,), lambda i: (0,)),
            pl.BlockSpec((d, dn), lambda i: (0, 0)),
        ],
        out_specs=pl.BlockSpec((blk, dn), lambda i: (i, 0)),
        out_shape=jax.ShapeDtypeStruct((n, dn), jnp.float32),
    )(aggp, g, dinv, b, w)


def _tc_final(aggp, g, dinv, b, n, blk):
    d = g.shape[1]
    dout = b.shape[0]

    def body(aggp_ref, g_ref, dinv_ref, b_ref, out_ref):
        s = aggp_ref[0] + aggp_ref[1] + g_ref[...]
        v = (s * dinv_ref[...])[:, :dout] + b_ref[...][None, :]
        m = jnp.max(v, axis=-1, keepdims=True)
        e = v - m
        out_ref[...] = e - jnp.log(jnp.sum(jnp.exp(e), axis=-1, keepdims=True))

    grid = (n // blk,)
    return pl.pallas_call(
        body,
        grid=grid,
        in_specs=[
            pl.BlockSpec((NC, blk, d), lambda i: (0, i, 0)),
            pl.BlockSpec((blk, d), lambda i: (i, 0)),
            pl.BlockSpec((blk, 1), lambda i: (i, 0)),
            pl.BlockSpec((dout,), lambda i: (0,)),
        ],
        out_specs=pl.BlockSpec((blk, dout), lambda i: (i, 0)),
        out_shape=jax.ShapeDtypeStruct((n, dout), jnp.float32),
    )(aggp, g, dinv, b)


def kernel(x, edge_index, W0, b0, W1, b1, W2, b2):
    n, din = x.shape
    e = edge_index.shape[1]

    np_rows = ((n + 16) + 16 * NW - 1) // (16 * NW) * (16 * NW)
    n_trash = np_rows - n

    e_chunks = e // C
    base, rem = e_chunks // NW, e_chunks % NW
    edge3 = edge_index.reshape(2, e_chunks, C).transpose(1, 0, 2)

    blk = 1000
    degp = _sc_degree(edge3, np_rows, base, rem)
    degsum = (degp[0] + degp[1]).reshape(np_rows, 1)
    dinv, g0 = _tc_scale(degsum, _tc_matmul(x, W0, n, blk), n, blk)
    a0 = _sc_aggregate(g0, edge3, np_rows, base, rem, W0.shape[1])
    g1 = _tc_mid(a0, g0, dinv, b0, W1, n, blk)
    a1 = _sc_aggregate(g1, edge3, np_rows, base, rem, W1.shape[1])
    W2p = jnp.pad(W2, ((0, 0), (0, 128 - W2.shape[1])))
    g2 = _tc_mid(a1, g1, dinv, b1, W2p, n, blk)
    a2 = _sc_aggregate(g2, edge3, np_rows, base, rem, W2p.shape[1])
    return _tc_final(a2, g2, dinv, b2, n, blk)


# speedup vs baseline: 1.1801x; 1.0188x over previous
import functools

import jax
import jax.numpy as jnp
from jax import lax
from jax.experimental import pallas as pl
from jax.experimental.pallas import tpu as pltpu
from jax.experimental.pallas import tpu_sc as plsc

NC = 2
NS = 16
NW = NC * NS
C = 128
GRP = 16


def _fill(ref, n, value):
    def body(i, _):
        ref[pl.ds(i * 16, 16)] = jnp.full((16,), value, jnp.float32)
        return 0
    lax.fori_loop(0, n // 16, body, 0)


def _fill2d(ref, rows, cols, value):
    def body(i, _):
        r = i // (cols // 16)
        c = i % (cols // 16)
        ref[r, pl.ds(c * 16, 16)] = jnp.full((16,), value, jnp.float32)
        return 0
    lax.fori_loop(0, rows * (cols // 16), body, 0)


def _sc_degree(edge3, np_rows, base, rem):
    rows_per_tile = np_rows // NS
    full_groups, tail = base // GRP, base % GRP
    mesh = plsc.VectorSubcoreMesh(core_axis_name="c", subcore_axis_name="s")

    @functools.partial(
        pl.kernel,
        out_type=jax.ShapeDtypeStruct((NC, np_rows), jnp.float32),
        mesh=mesh,
        scratch_types=[
            pltpu.VMEM_SHARED((np_rows,), jnp.float32),
            pltpu.VMEM((GRP, 2, C), jnp.int32),
            pltpu.VMEM((C,), jnp.float32),
            pltpu.VMEM((rows_per_tile,), jnp.float32),
        ],
    )
    def deg_kernel(edge_hbm, out_hbm, acc, idx_v, ones_v, zeros_v):
        cid = lax.axis_index("c")
        sid = lax.axis_index("s")
        wid = cid * NS + sid
        start = base * wid + jnp.minimum(wid, rem)
        _fill(ones_v, C, 1.0)
        _fill(zeros_v, rows_per_tile, 0.0)
        pltpu.sync_copy(zeros_v, acc.at[pl.ds(sid * rows_per_tile, rows_per_tile)])
        plsc.subcore_barrier()

        def run(nchunks):
            def chunk(j, _):
                pltpu.sync_copy(ones_v, acc.at[idx_v.at[j, 1]], add=True)
                return 0
            lax.fori_loop(0, nchunks, chunk, 0)

        def group(gi, _):
            pltpu.sync_copy(edge_hbm.at[pl.ds(start + gi * GRP, GRP)], idx_v)
            run(GRP)
            return 0
        lax.fori_loop(0, full_groups, group, 0)
        if tail:
            pltpu.sync_copy(
                edge_hbm.at[pl.ds(start + full_groups * GRP, tail)],
                idx_v.at[pl.ds(0, tail)])
            run(tail)
        if rem:
            @pl.when(wid < rem)
            def _():
                pltpu.sync_copy(edge_hbm.at[pl.ds(start + base, 1)],
                                idx_v.at[pl.ds(0, 1)])
                pltpu.sync_copy(ones_v, acc.at[idx_v.at[0, 1]], add=True)
        plsc.subcore_barrier()
        pltpu.sync_copy(acc.at[pl.ds(sid * rows_per_tile, rows_per_tile)],
                        out_hbm.at[cid, pl.ds(sid * rows_per_tile, rows_per_tile)])

    return deg_kernel(edge3)


def _sc_aggregate(g, edge3, np_rows, base, rem, d):
    rows_per_tile = np_rows // NS
    full_groups, tail = base // GRP, base % GRP
    mesh = plsc.VectorSubcoreMesh(core_axis_name="c", subcore_axis_name="s")

    @functools.partial(
        pl.kernel,
        out_type=jax.ShapeDtypeStruct((NC, np_rows, d), jnp.float32),
        mesh=mesh,
        scratch_types=[
            pltpu.VMEM_SHARED((np_rows, d), jnp.float32),
            pltpu.VMEM((2, GRP, 2, C), jnp.int32),
            pltpu.VMEM((2, C, d), jnp.float32),
            pltpu.SemaphoreType.DMA,
            pltpu.SemaphoreType.DMA,
            pltpu.SemaphoreType.DMA,
        ],
    )
    def agg_kernel(g_hbm, edge_hbm, out_hbm, acc, idx_v, rows_v,
                   sem0, sem1, sem_i):
        cid = lax.axis_index("c")
        sid = lax.axis_index("s")
        wid = cid * NS + sid
        start = base * wid + jnp.minimum(wid, rem)

        _fill2d(rows_v.at[0], C, d, 0.0)
        for k in range(rows_per_tile // C):
            pltpu.sync_copy(rows_v.at[0],
                            acc.at[pl.ds(sid * rows_per_tile + k * C, C)])
        plsc.subcore_barrier()

        def wait_gather(buf, sem):
            pltpu.make_async_copy(g_hbm.at[pl.ds(0, C)], buf, sem).wait()

        def pairs(iv, npairs):
            pltpu.async_copy(g_hbm.at[iv.at[0, 0]], rows_v.at[0], sem0)

            def pair(t, _):
                pltpu.async_copy(g_hbm.at[iv.at[2 * t + 1, 0]], rows_v.at[1],
                                 sem1)
                wait_gather(rows_v.at[0], sem0)
                pltpu.sync_copy(rows_v.at[0], acc.at[iv.at[2 * t, 1]],
                                add=True)

                @pl.when(t + 1 < npairs)
                def _():
                    pltpu.async_copy(g_hbm.at[iv.at[2 * t + 2, 0]],
                                     rows_v.at[0], sem0)
                wait_gather(rows_v.at[1], sem1)
                pltpu.sync_copy(rows_v.at[1], acc.at[iv.at[2 * t + 1, 1]],
                                add=True)
                return 0
            lax.fori_loop(0, npairs, pair, 0)

        pltpu.sync_copy(edge_hbm.at[pl.ds(start, GRP)], idx_v.at[0])

        def group(gi, _):
            s = gi % 2
            iv = idx_v.at[s]

            @pl.when(gi > 0)
            def _():
                pltpu.make_async_copy(edge_hbm.at[pl.ds(0, GRP)], iv,
                                      sem_i).wait()

            @pl.when(gi + 1 < full_groups)
            def _():
                pltpu.async_copy(
                    edge_hbm.at[pl.ds(start + (gi + 1) * GRP, GRP)],
                    idx_v.at[1 - s], sem_i)
            if tail:
                @pl.when(gi + 1 == full_groups)
                def _():
                    pltpu.async_copy(
                        edge_hbm.at[pl.ds(start + full_groups * GRP, tail)],
                        idx_v.at[1 - s].at[pl.ds(0, tail)], sem_i)
            pairs(iv, GRP // 2)
            return 0
        lax.fori_loop(0, full_groups, group, 0)

        if tail:
            s = full_groups % 2
            iv = idx_v.at[s]
            if full_groups:
                pltpu.make_async_copy(edge_hbm.at[pl.ds(0, tail)],
                                      iv.at[pl.ds(0, tail)], sem_i).wait()
            else:
                pltpu.sync_copy(edge_hbm.at[pl.ds(start, tail)],
                                iv.at[pl.ds(0, tail)])
            pairs(iv, tail // 2)
            if tail % 2:
                pltpu.async_copy(g_hbm.at[iv.at[tail - 1, 0]], rows_v.at[0],
                                 sem0)
                wait_gather(rows_v.at[0], sem0)
                pltpu.sync_copy(rows_v.at[0], acc.at[iv.at[tail - 1, 1]],
                                add=True)
        if rem:
            @pl.when(wid < rem)
            def _():
                pltpu.sync_copy(edge_hbm.at[pl.ds(start + base, 1)],
                                idx_v.at[0].at[pl.ds(0, 1)])
                pltpu.async_copy(g_hbm.at[idx_v.at[0, 0, 0]], rows_v.at[0],
                                 sem0)
                wait_gather(rows_v.at[0], sem0)
                pltpu.sync_copy(rows_v.at[0], acc.at[idx_v.at[0, 0, 1]],
                                add=True)
        plsc.subcore_barrier()
        pltpu.sync_copy(acc.at[pl.ds(sid * rows_per_tile, rows_per_tile)],
                        out_hbm.at[cid, pl.ds(sid * rows_per_tile, rows_per_tile)])

    return agg_kernel(g, edge3)


def _tc_matmul(x, w0, n, blk):
    din, dh = w0.shape

    def body(x_ref, w_ref, h_ref):
        h_ref[...] = jnp.dot(x_ref[...], w_ref[...],
                             preferred_element_type=jnp.float32)

    grid = (n // blk,)
    return pl.pallas_call(
        body,
        grid=grid,
        in_specs=[
            pl.BlockSpec((blk, din), lambda i: (i, 0)),
            pl.BlockSpec((din, dh), lambda i: (0, 0)),
        ],
        out_specs=pl.BlockSpec((blk, dh), lambda i: (i, 0)),
        out_shape=jax.ShapeDtypeStruct((n, dh), jnp.float32),
    )(x, w0)


def _tc_scale(degsum, h0, n, blk):
    dh = h0.shape[1]

    def body(deg_ref, h_ref, dinv_ref, g_ref):
        dv = lax.rsqrt(deg_ref[...] + 1.0)
        dinv_ref[...] = dv
        g_ref[...] = h_ref[...] * dv

    grid = (n // blk,)
    return pl.pallas_call(
        body,
        grid=grid,
        in_specs=[
            pl.BlockSpec((blk, 1), lambda i: (i, 0)),
            pl.BlockSpec((blk, dh), lambda i: (i, 0)),
        ],
        out_specs=[
            pl.BlockSpec((blk, 1), lambda i: (i, 0)),
            pl.BlockSpec((blk, dh), lambda i: (i, 0)),
        ],
        out_shape=[
            jax.ShapeDtypeStruct((n, 1), jnp.float32),
            jax.ShapeDtypeStruct((n, dh), jnp.float32),
        ],
    )(degsum, h0)


def _tc_mid(aggp, g, dinv, b, w, n, blk):
    d, dn = w.shape

    def body(aggp_ref, g_ref, dinv_ref, b_ref, w_ref, out_ref):
        s = aggp_ref[0] + aggp_ref[1] + g_ref[...]
        dv = dinv_ref[...]
        h = jnp.maximum(s * dv + b_ref[...][None, :], 0.0)
        out_ref[...] = jnp.dot(h, w_ref[...],
                               preferred_element_type=jnp.float32) * dv

    grid = (n // blk,)
    return pl.pallas_call(
        body,
        grid=grid,
        in_specs=[
            pl.BlockSpec((NC, blk, d), lambda i: (0, i, 0)),
            pl.BlockSpec((blk, d), lambda i: (i, 0)),
            pl.BlockSpec((blk, 1), lambda i: (i, 0)),
            pl.BlockSpec((d,), lambda i: (0,)),
            pl.BlockSpec((d, dn), lambda i: (0, 0)),
        ],
        out_specs=pl.BlockSpec((blk, dn), lambda i: (i, 0)),
        out_shape=jax.ShapeDtypeStruct((n, dn), jnp.float32),
    )(aggp, g, dinv, b, w)


def _tc_final(aggp, g, dinv, b, n, blk):
    d = g.shape[1]
    dout = b.shape[0]

    def body(aggp_ref, g_ref, dinv_ref, b_ref, out_ref):
        s = aggp_ref[0] + aggp_ref[1] + g_ref[...]
        v = (s * dinv_ref[...])[:, :dout] + b_ref[...][None, :]
        m = jnp.max(v, axis=-1, keepdims=True)
        e = v - m
        out_ref[...] = e - jnp.log(jnp.sum(jnp.exp(e), axis=-1, keepdims=True))

    grid = (n // blk,)
    return pl.pallas_call(
        body,
        grid=grid,
        in_specs=[
            pl.BlockSpec((NC, blk, d), lambda i: (0, i, 0)),
            pl.BlockSpec((blk, d), lambda i: (i, 0)),
            pl.BlockSpec((blk, 1), lambda i: (i, 0)),
            pl.BlockSpec((dout,), lambda i: (0,)),
        ],
        out_specs=pl.BlockSpec((blk, dout), lambda i: (i, 0)),
        out_shape=jax.ShapeDtypeStruct((n, dout), jnp.float32),
    )(aggp, g, dinv, b)


def kernel(x, edge_index, W0, b0, W1, b1, W2, b2):
    n, din = x.shape
    e = edge_index.shape[1]

    np_rows = ((n + 16) + 16 * NW - 1) // (16 * NW) * (16 * NW)
    n_trash = np_rows - n

    e_chunks = e // C
    base, rem = e_chunks // NW, e_chunks % NW
    edge3 = edge_index.reshape(2, e_chunks, C).transpose(1, 0, 2)

    blk = 2000
    degp = _sc_degree(edge3, np_rows, base, rem)
    degsum = (degp[0] + degp[1]).reshape(np_rows, 1)
    dinv, g0 = _tc_scale(degsum, _tc_matmul(x, W0, n, blk), n, blk)
    a0 = _sc_aggregate(g0, edge3, np_rows, base, rem, W0.shape[1])
    g1 = _tc_mid(a0, g0, dinv, b0, W1, n, blk)
    a1 = _sc_aggregate(g1, edge3, np_rows, base, rem, W1.shape[1])
    W2p = jnp.pad(W2, ((0, 0), (0, 128 - W2.shape[1])))
    g2 = _tc_mid(a1, g1, dinv, b1, W2p, n, blk)
    a2 = _sc_aggregate(g2, edge3, np_rows, base, rem, W2p.shape[1])
    return _tc_final(a2, g2, dinv, b2, n, blk)


# speedup vs baseline: 1.1894x; 1.0079x over previous
import functools

import jax
import jax.numpy as jnp
from jax import lax
from jax.experimental import pallas as pl
from jax.experimental.pallas import tpu as pltpu
from jax.experimental.pallas import tpu_sc as plsc

NC = 2
NS = 16
NW = NC * NS
C = 128
GRP = 16


def _fill(ref, n, value):
    def body(i, _):
        ref[pl.ds(i * 16, 16)] = jnp.full((16,), value, jnp.float32)
        return 0
    lax.fori_loop(0, n // 16, body, 0)


def _fill2d(ref, rows, cols, value):
    def body(i, _):
        r = i // (cols // 16)
        c = i % (cols // 16)
        ref[r, pl.ds(c * 16, 16)] = jnp.full((16,), value, jnp.float32)
        return 0
    lax.fori_loop(0, rows * (cols // 16), body, 0)


def _sc_degree(edge3, np_rows, base, rem):
    rows_per_tile = np_rows // NS
    full_groups, tail = base // GRP, base % GRP
    mesh = plsc.VectorSubcoreMesh(core_axis_name="c", subcore_axis_name="s")

    @functools.partial(
        pl.kernel,
        out_type=jax.ShapeDtypeStruct((NC, np_rows), jnp.float32),
        mesh=mesh,
        scratch_types=[
            pltpu.VMEM_SHARED((np_rows,), jnp.float32),
            pltpu.VMEM((GRP, 2, C), jnp.int32),
            pltpu.VMEM((C,), jnp.float32),
            pltpu.VMEM((rows_per_tile,), jnp.float32),
        ],
    )
    def deg_kernel(edge_hbm, out_hbm, acc, idx_v, ones_v, zeros_v):
        cid = lax.axis_index("c")
        sid = lax.axis_index("s")
        wid = cid * NS + sid
        start = base * wid + jnp.minimum(wid, rem)
        _fill(ones_v, C, 1.0)
        _fill(zeros_v, rows_per_tile, 0.0)
        pltpu.sync_copy(zeros_v, acc.at[pl.ds(sid * rows_per_tile, rows_per_tile)])
        plsc.subcore_barrier()

        def run(nchunks):
            def chunk(j, _):
                pltpu.sync_copy(ones_v, acc.at[idx_v.at[j, 1]], add=True)
                return 0
            lax.fori_loop(0, nchunks, chunk, 0)

        def group(gi, _):
            pltpu.sync_copy(edge_hbm.at[pl.ds(start + gi * GRP, GRP)], idx_v)
            run(GRP)
            return 0
        lax.fori_loop(0, full_groups, group, 0)
        if tail:
            pltpu.sync_copy(
                edge_hbm.at[pl.ds(start + full_groups * GRP, tail)],
                idx_v.at[pl.ds(0, tail)])
            run(tail)
        if rem:
            @pl.when(wid < rem)
            def _():
                pltpu.sync_copy(edge_hbm.at[pl.ds(start + base, 1)],
                                idx_v.at[pl.ds(0, 1)])
                pltpu.sync_copy(ones_v, acc.at[idx_v.at[0, 1]], add=True)
        plsc.subcore_barrier()
        pltpu.sync_copy(acc.at[pl.ds(sid * rows_per_tile, rows_per_tile)],
                        out_hbm.at[cid, pl.ds(sid * rows_per_tile, rows_per_tile)])

    return deg_kernel(edge3)


def _sc_aggregate(g, edge3, np_rows, base, rem, d):
    rows_per_tile = np_rows // NS
    full_groups, tail = base // GRP, base % GRP
    mesh = plsc.VectorSubcoreMesh(core_axis_name="c", subcore_axis_name="s")

    @functools.partial(
        pl.kernel,
        out_type=jax.ShapeDtypeStruct((NC, np_rows, d), jnp.float32),
        mesh=mesh,
        scratch_types=[
            pltpu.VMEM_SHARED((np_rows, d), jnp.float32),
            pltpu.VMEM((2, GRP, 2, C), jnp.int32),
            pltpu.VMEM((2, C, d), jnp.float32),
            pltpu.SemaphoreType.DMA,
            pltpu.SemaphoreType.DMA,
            pltpu.SemaphoreType.DMA,
        ],
    )
    def agg_kernel(g_hbm, edge_hbm, out_hbm, acc, idx_v, rows_v,
                   sem0, sem1, sem_i):
        cid = lax.axis_index("c")
        sid = lax.axis_index("s")
        wid = cid * NS + sid
        start = base * wid + jnp.minimum(wid, rem)

        _fill2d(rows_v.at[0], C, d, 0.0)
        for k in range(rows_per_tile // C):
            pltpu.sync_copy(rows_v.at[0],
                            acc.at[pl.ds(sid * rows_per_tile + k * C, C)])
        plsc.subcore_barrier()

        def wait_gather(buf, sem):
            pltpu.make_async_copy(g_hbm.at[pl.ds(0, C)], buf, sem).wait()

        def pairs(iv, npairs):
            pltpu.async_copy(g_hbm.at[iv.at[0, 0]], rows_v.at[0], sem0)

            def pair(t, _):
                pltpu.async_copy(g_hbm.at[iv.at[2 * t + 1, 0]], rows_v.at[1],
                                 sem1)
                wait_gather(rows_v.at[0], sem0)
                pltpu.sync_copy(rows_v.at[0], acc.at[iv.at[2 * t, 1]],
                                add=True)

                @pl.when(t + 1 < npairs)
                def _():
                    pltpu.async_copy(g_hbm.at[iv.at[2 * t + 2, 0]],
                                     rows_v.at[0], sem0)
                wait_gather(rows_v.at[1], sem1)
                pltpu.sync_copy(rows_v.at[1], acc.at[iv.at[2 * t + 1, 1]],
                                add=True)
                return 0
            lax.fori_loop(0, npairs, pair, 0)

        pltpu.sync_copy(edge_hbm.at[pl.ds(start, GRP)], idx_v.at[0])

        def group(gi, _):
            s = gi % 2
            iv = idx_v.at[s]

            @pl.when(gi > 0)
            def _():
                pltpu.make_async_copy(edge_hbm.at[pl.ds(0, GRP)], iv,
                                      sem_i).wait()

            @pl.when(gi + 1 < full_groups)
            def _():
                pltpu.async_copy(
                    edge_hbm.at[pl.ds(start + (gi + 1) * GRP, GRP)],
                    idx_v.at[1 - s], sem_i)
            if tail:
                @pl.when(gi + 1 == full_groups)
                def _():
                    pltpu.async_copy(
                        edge_hbm.at[pl.ds(start + full_groups * GRP, tail)],
                        idx_v.at[1 - s].at[pl.ds(0, tail)], sem_i)
            pairs(iv, GRP // 2)
            return 0
        lax.fori_loop(0, full_groups, group, 0)

        if tail:
            s = full_groups % 2
            iv = idx_v.at[s]
            if full_groups:
                pltpu.make_async_copy(edge_hbm.at[pl.ds(0, tail)],
                                      iv.at[pl.ds(0, tail)], sem_i).wait()
            else:
                pltpu.sync_copy(edge_hbm.at[pl.ds(start, tail)],
                                iv.at[pl.ds(0, tail)])
            pairs(iv, tail // 2)
            if tail % 2:
                pltpu.async_copy(g_hbm.at[iv.at[tail - 1, 0]], rows_v.at[0],
                                 sem0)
                wait_gather(rows_v.at[0], sem0)
                pltpu.sync_copy(rows_v.at[0], acc.at[iv.at[tail - 1, 1]],
                                add=True)
        if rem:
            @pl.when(wid < rem)
            def _():
                pltpu.sync_copy(edge_hbm.at[pl.ds(start + base, 1)],
                                idx_v.at[0].at[pl.ds(0, 1)])
                pltpu.async_copy(g_hbm.at[idx_v.at[0, 0, 0]], rows_v.at[0],
                                 sem0)
                wait_gather(rows_v.at[0], sem0)
                pltpu.sync_copy(rows_v.at[0], acc.at[idx_v.at[0, 0, 1]],
                                add=True)
        plsc.subcore_barrier()
        pltpu.sync_copy(acc.at[pl.ds(sid * rows_per_tile, rows_per_tile)],
                        out_hbm.at[cid, pl.ds(sid * rows_per_tile, rows_per_tile)])

    return agg_kernel(g, edge3)


def _tc_matmul(x, w0, n, blk):
    din, dh = w0.shape

    def body(x_ref, w_ref, h_ref):
        h_ref[...] = jnp.dot(x_ref[...], w_ref[...],
                             preferred_element_type=jnp.float32)

    grid = (n // blk,)
    return pl.pallas_call(
        body,
        grid=grid,
        in_specs=[
            pl.BlockSpec((blk, din), lambda i: (i, 0)),
            pl.BlockSpec((din, dh), lambda i: (0, 0)),
        ],
        out_specs=pl.BlockSpec((blk, dh), lambda i: (i, 0)),
        out_shape=jax.ShapeDtypeStruct((n, dh), jnp.float32),
    )(x, w0)


def _tc_scale(degsum, h0, n, blk):
    dh = h0.shape[1]

    def body(deg_ref, h_ref, dinv_ref, g_ref):
        dv = lax.rsqrt(deg_ref[...] + 1.0)
        dinv_ref[...] = dv
        g_ref[...] = h_ref[...] * dv

    grid = (n // blk,)
    return pl.pallas_call(
        body,
        grid=grid,
        in_specs=[
            pl.BlockSpec((blk, 1), lambda i: (i, 0)),
            pl.BlockSpec((blk, dh), lambda i: (i, 0)),
        ],
        out_specs=[
            pl.BlockSpec((blk, 1), lambda i: (i, 0)),
            pl.BlockSpec((blk, dh), lambda i: (i, 0)),
        ],
        out_shape=[
            jax.ShapeDtypeStruct((n, 1), jnp.float32),
            jax.ShapeDtypeStruct((n, dh), jnp.float32),
        ],
    )(degsum, h0)


def _tc_mid(aggp, g, dinv, b, w, n, blk):
    d, dn = w.shape

    def body(aggp_ref, g_ref, dinv_ref, b_ref, w_ref, out_ref):
        s = aggp_ref[0] + aggp_ref[1] + g_ref[...]
        dv = dinv_ref[...]
        h = jnp.maximum(s * dv + b_ref[...][None, :], 0.0)
        out_ref[...] = jnp.dot(h, w_ref[...],
                               preferred_element_type=jnp.float32) * dv

    grid = (n // blk,)
    return pl.pallas_call(
        body,
        grid=grid,
        in_specs=[
            pl.BlockSpec((NC, blk, d), lambda i: (0, i, 0)),
            pl.BlockSpec((blk, d), lambda i: (i, 0)),
            pl.BlockSpec((blk, 1), lambda i: (i, 0)),
            pl.BlockSpec((d,), lambda i: (0,)),
            pl.BlockSpec((d, dn), lambda i: (0, 0)),
        ],
        out_specs=pl.BlockSpec((blk, dn), lambda i: (i, 0)),
        out_shape=jax.ShapeDtypeStruct((n, dn), jnp.float32),
    )(aggp, g, dinv, b, w)


def _tc_final(aggp, g, dinv, b, n, blk):
    d = g.shape[1]
    dout = b.shape[0]

    def body(aggp_ref, g_ref, dinv_ref, b_ref, out_ref):
        s = aggp_ref[0] + aggp_ref[1] + g_ref[...]
        v = (s * dinv_ref[...])[:, :dout] + b_ref[...][None, :]
        m = jnp.max(v, axis=-1, keepdims=True)
        e = v - m
        out_ref[...] = e - jnp.log(jnp.sum(jnp.exp(e), axis=-1, keepdims=True))

    grid = (n // blk,)
    return pl.pallas_call(
        body,
        grid=grid,
        in_specs=[
            pl.BlockSpec((NC, blk, d), lambda i: (0, i, 0)),
            pl.BlockSpec((blk, d), lambda i: (i, 0)),
            pl.BlockSpec((blk, 1), lambda i: (i, 0)),
            pl.BlockSpec((dout,), lambda i: (0,)),
        ],
        out_specs=pl.BlockSpec((blk, dout), lambda i: (i, 0)),
        out_shape=jax.ShapeDtypeStruct((n, dout), jnp.float32),
    )(aggp, g, dinv, b)


def kernel(x, edge_index, W0, b0, W1, b1, W2, b2):
    n, din = x.shape
    e = edge_index.shape[1]

    np_rows = ((n + 16) + 16 * NW - 1) // (16 * NW) * (16 * NW)
    n_trash = np_rows - n

    e_chunks = e // C
    base, rem = e_chunks // NW, e_chunks % NW
    edge3 = edge_index.reshape(2, e_chunks, C).transpose(1, 0, 2)

    blk = 5000
    degp = _sc_degree(edge3, np_rows, base, rem)
    degsum = (degp[0] + degp[1]).reshape(np_rows, 1)
    dinv, g0 = _tc_scale(degsum, _tc_matmul(x, W0, n, blk), n, blk)
    a0 = _sc_aggregate(g0, edge3, np_rows, base, rem, W0.shape[1])
    g1 = _tc_mid(a0, g0, dinv, b0, W1, n, blk)
    a1 = _sc_aggregate(g1, edge3, np_rows, base, rem, W1.shape[1])
    W2p = jnp.pad(W2, ((0, 0), (0, 128 - W2.shape[1])))
    g2 = _tc_mid(a1, g1, dinv, b1, W2p, n, blk)
    a2 = _sc_aggregate(g2, edge3, np_rows, base, rem, W2p.shape[1])
    return _tc_final(a2, g2, dinv, b2, n, blk)


# speedup vs baseline: 1.1948x; 1.0045x over previous
import functools

import jax
import jax.numpy as jnp
from jax import lax
from jax.experimental import pallas as pl
from jax.experimental.pallas import tpu as pltpu
from jax.experimental.pallas import tpu_sc as plsc

NC = 2
NS = 16
NW = NC * NS
C = 128
GRP = 16


def _fill(ref, n, value):
    def body(i, _):
        ref[pl.ds(i * 16, 16)] = jnp.full((16,), value, jnp.float32)
        return 0
    lax.fori_loop(0, n // 16, body, 0)


def _fill2d(ref, rows, cols, value):
    def body(i, _):
        r = i // (cols // 16)
        c = i % (cols // 16)
        ref[r, pl.ds(c * 16, 16)] = jnp.full((16,), value, jnp.float32)
        return 0
    lax.fori_loop(0, rows * (cols // 16), body, 0)


def _sc_degree(edge3, np_rows, base, rem):
    rows_per_tile = np_rows // NS
    full_groups, tail = base // GRP, base % GRP
    mesh = plsc.VectorSubcoreMesh(core_axis_name="c", subcore_axis_name="s")

    @functools.partial(
        pl.kernel,
        out_type=jax.ShapeDtypeStruct((NC, np_rows), jnp.float32),
        mesh=mesh,
        scratch_types=[
            pltpu.VMEM_SHARED((np_rows,), jnp.float32),
            pltpu.VMEM((GRP, 2, C), jnp.int32),
            pltpu.VMEM((C,), jnp.float32),
            pltpu.VMEM((rows_per_tile,), jnp.float32),
        ],
    )
    def deg_kernel(edge_hbm, out_hbm, acc, idx_v, ones_v, zeros_v):
        cid = lax.axis_index("c")
        sid = lax.axis_index("s")
        wid = cid * NS + sid
        start = base * wid + jnp.minimum(wid, rem)
        _fill(ones_v, C, 1.0)
        _fill(zeros_v, rows_per_tile, 0.0)
        pltpu.sync_copy(zeros_v, acc.at[pl.ds(sid * rows_per_tile, rows_per_tile)])
        plsc.subcore_barrier()

        def run(nchunks):
            def chunk(j, _):
                pltpu.sync_copy(ones_v, acc.at[idx_v.at[j, 1]], add=True)
                return 0
            lax.fori_loop(0, nchunks, chunk, 0)

        def group(gi, _):
            pltpu.sync_copy(edge_hbm.at[pl.ds(start + gi * GRP, GRP)], idx_v)
            run(GRP)
            return 0
        lax.fori_loop(0, full_groups, group, 0)
        if tail:
            pltpu.sync_copy(
                edge_hbm.at[pl.ds(start + full_groups * GRP, tail)],
                idx_v.at[pl.ds(0, tail)])
            run(tail)
        if rem:
            @pl.when(wid < rem)
            def _():
                pltpu.sync_copy(edge_hbm.at[pl.ds(start + base, 1)],
                                idx_v.at[pl.ds(0, 1)])
                pltpu.sync_copy(ones_v, acc.at[idx_v.at[0, 1]], add=True)
        plsc.subcore_barrier()
        pltpu.sync_copy(acc.at[pl.ds(sid * rows_per_tile, rows_per_tile)],
                        out_hbm.at[cid, pl.ds(sid * rows_per_tile, rows_per_tile)])

    return deg_kernel(edge3)


def _sc_aggregate(g, edge3, np_rows, base, rem, d):
    rows_per_tile = np_rows // NS
    full_groups, tail = base // GRP, base % GRP
    mesh = plsc.VectorSubcoreMesh(core_axis_name="c", subcore_axis_name="s")

    @functools.partial(
        pl.kernel,
        out_type=jax.ShapeDtypeStruct((NC, np_rows, d), jnp.float32),
        mesh=mesh,
        scratch_types=[
            pltpu.VMEM_SHARED((np_rows, d), jnp.float32),
            pltpu.VMEM((2, GRP, 2, C), jnp.int32),
            pltpu.VMEM((2, C, d), jnp.float32),
            pltpu.SemaphoreType.DMA,
            pltpu.SemaphoreType.DMA,
            pltpu.SemaphoreType.DMA,
        ],
    )
    def agg_kernel(g_hbm, edge_hbm, out_hbm, acc, idx_v, rows_v,
                   sem0, sem1, sem_i):
        cid = lax.axis_index("c")
        sid = lax.axis_index("s")
        wid = cid * NS + sid
        start = base * wid + jnp.minimum(wid, rem)

        _fill2d(rows_v.at[0], C, d, 0.0)
        for k in range(rows_per_tile // C):
            pltpu.async_copy(rows_v.at[0],
                             acc.at[pl.ds(sid * rows_per_tile + k * C, C)],
                             sem_i)
        for k in range(rows_per_tile // C):
            pltpu.make_async_copy(
                rows_v.at[0],
                acc.at[pl.ds(sid * rows_per_tile + k * C, C)],
                sem_i).wait()
        plsc.subcore_barrier()

        def wait_gather(buf, sem):
            pltpu.make_async_copy(g_hbm.at[pl.ds(0, C)], buf, sem).wait()

        def pairs(iv, npairs):
            pltpu.async_copy(g_hbm.at[iv.at[0, 0]], rows_v.at[0], sem0)

            def pair(t, _):
                pltpu.async_copy(g_hbm.at[iv.at[2 * t + 1, 0]], rows_v.at[1],
                                 sem1)
                wait_gather(rows_v.at[0], sem0)
                pltpu.sync_copy(rows_v.at[0], acc.at[iv.at[2 * t, 1]],
                                add=True)

                @pl.when(t + 1 < npairs)
                def _():
                    pltpu.async_copy(g_hbm.at[iv.at[2 * t + 2, 0]],
                                     rows_v.at[0], sem0)
                wait_gather(rows_v.at[1], sem1)
                pltpu.sync_copy(rows_v.at[1], acc.at[iv.at[2 * t + 1, 1]],
                                add=True)
                return 0
            lax.fori_loop(0, npairs, pair, 0)

        pltpu.sync_copy(edge_hbm.at[pl.ds(start, GRP)], idx_v.at[0])

        def group(gi, _):
            s = gi % 2
            iv = idx_v.at[s]

            @pl.when(gi > 0)
            def _():
                pltpu.make_async_copy(edge_hbm.at[pl.ds(0, GRP)], iv,
                                      sem_i).wait()

            @pl.when(gi + 1 < full_groups)
            def _():
                pltpu.async_copy(
                    edge_hbm.at[pl.ds(start + (gi + 1) * GRP, GRP)],
                    idx_v.at[1 - s], sem_i)
            if tail:
                @pl.when(gi + 1 == full_groups)
                def _():
                    pltpu.async_copy(
                        edge_hbm.at[pl.ds(start + full_groups * GRP, tail)],
                        idx_v.at[1 - s].at[pl.ds(0, tail)], sem_i)
            pairs(iv, GRP // 2)
            return 0
        lax.fori_loop(0, full_groups, group, 0)

        if tail:
            s = full_groups % 2
            iv = idx_v.at[s]
            if full_groups:
                pltpu.make_async_copy(edge_hbm.at[pl.ds(0, tail)],
                                      iv.at[pl.ds(0, tail)], sem_i).wait()
            else:
                pltpu.sync_copy(edge_hbm.at[pl.ds(start, tail)],
                                iv.at[pl.ds(0, tail)])
            pairs(iv, tail // 2)
            if tail % 2:
                pltpu.async_copy(g_hbm.at[iv.at[tail - 1, 0]], rows_v.at[0],
                                 sem0)
                wait_gather(rows_v.at[0], sem0)
                pltpu.sync_copy(rows_v.at[0], acc.at[iv.at[tail - 1, 1]],
                                add=True)
        if rem:
            @pl.when(wid < rem)
            def _():
                pltpu.sync_copy(edge_hbm.at[pl.ds(start + base, 1)],
                                idx_v.at[0].at[pl.ds(0, 1)])
                pltpu.async_copy(g_hbm.at[idx_v.at[0, 0, 0]], rows_v.at[0],
                                 sem0)
                wait_gather(rows_v.at[0], sem0)
                pltpu.sync_copy(rows_v.at[0], acc.at[idx_v.at[0, 0, 1]],
                                add=True)
        plsc.subcore_barrier()
        pltpu.sync_copy(acc.at[pl.ds(sid * rows_per_tile, rows_per_tile)],
                        out_hbm.at[cid, pl.ds(sid * rows_per_tile, rows_per_tile)])

    return agg_kernel(g, edge3)


def _tc_matmul(x, w0, n, blk):
    din, dh = w0.shape

    def body(x_ref, w_ref, h_ref):
        h_ref[...] = jnp.dot(x_ref[...], w_ref[...],
                             preferred_element_type=jnp.float32)

    grid = (n // blk,)
    return pl.pallas_call(
        body,
        grid=grid,
        in_specs=[
            pl.BlockSpec((blk, din), lambda i: (i, 0)),
            pl.BlockSpec((din, dh), lambda i: (0, 0)),
        ],
        out_specs=pl.BlockSpec((blk, dh), lambda i: (i, 0)),
        out_shape=jax.ShapeDtypeStruct((n, dh), jnp.float32),
    )(x, w0)


def _tc_scale(degsum, h0, n, blk):
    dh = h0.shape[1]

    def body(deg_ref, h_ref, dinv_ref, g_ref):
        dv = lax.rsqrt(deg_ref[...] + 1.0)
        dinv_ref[...] = dv
        g_ref[...] = h_ref[...] * dv

    grid = (n // blk,)
    return pl.pallas_call(
        body,
        grid=grid,
        in_specs=[
            pl.BlockSpec((blk, 1), lambda i: (i, 0)),
            pl.BlockSpec((blk, dh), lambda i: (i, 0)),
        ],
        out_specs=[
            pl.BlockSpec((blk, 1), lambda i: (i, 0)),
            pl.BlockSpec((blk, dh), lambda i: (i, 0)),
        ],
        out_shape=[
            jax.ShapeDtypeStruct((n, 1), jnp.float32),
            jax.ShapeDtypeStruct((n, dh), jnp.float32),
        ],
    )(degsum, h0)


def _tc_mid(aggp, g, dinv, b, w, n, blk):
    d, dn = w.shape

    def body(aggp_ref, g_ref, dinv_ref, b_ref, w_ref, out_ref):
        s = aggp_ref[0] + aggp_ref[1] + g_ref[...]
        dv = dinv_ref[...]
        h = jnp.maximum(s * dv + b_ref[...][None, :], 0.0)
        out_ref[...] = jnp.dot(h, w_ref[...],
                               preferred_element_type=jnp.float32) * dv

    grid = (n // blk,)
    return pl.pallas_call(
        body,
        grid=grid,
        in_specs=[
            pl.BlockSpec((NC, blk, d), lambda i: (0, i, 0)),
            pl.BlockSpec((blk, d), lambda i: (i, 0)),
            pl.BlockSpec((blk, 1), lambda i: (i, 0)),
            pl.BlockSpec((d,), lambda i: (0,)),
            pl.BlockSpec((d, dn), lambda i: (0, 0)),
        ],
        out_specs=pl.BlockSpec((blk, dn), lambda i: (i, 0)),
        out_shape=jax.ShapeDtypeStruct((n, dn), jnp.float32),
    )(aggp, g, dinv, b, w)


def _tc_final(aggp, g, dinv, b, n, blk):
    d = g.shape[1]
    dout = b.shape[0]

    def body(aggp_ref, g_ref, dinv_ref, b_ref, out_ref):
        s = aggp_ref[0] + aggp_ref[1] + g_ref[...]
        v = (s * dinv_ref[...])[:, :dout] + b_ref[...][None, :]
        m = jnp.max(v, axis=-1, keepdims=True)
        e = v - m
        out_ref[...] = e - jnp.log(jnp.sum(jnp.exp(e), axis=-1, keepdims=True))

    grid = (n // blk,)
    return pl.pallas_call(
        body,
        grid=grid,
        in_specs=[
            pl.BlockSpec((NC, blk, d), lambda i: (0, i, 0)),
            pl.BlockSpec((blk, d), lambda i: (i, 0)),
            pl.BlockSpec((blk, 1), lambda i: (i, 0)),
            pl.BlockSpec((dout,), lambda i: (0,)),
        ],
        out_specs=pl.BlockSpec((blk, dout), lambda i: (i, 0)),
        out_shape=jax.ShapeDtypeStruct((n, dout), jnp.float32),
    )(aggp, g, dinv, b)


def kernel(x, edge_index, W0, b0, W1, b1, W2, b2):
    n, din = x.shape
    e = edge_index.shape[1]

    np_rows = ((n + 16) + 16 * NW - 1) // (16 * NW) * (16 * NW)
    n_trash = np_rows - n

    e_chunks = e // C
    base, rem = e_chunks // NW, e_chunks % NW
    edge3 = edge_index.reshape(2, e_chunks, C).transpose(1, 0, 2)

    blk = 5000
    degp = _sc_degree(edge3, np_rows, base, rem)
    degsum = (degp[0] + degp[1]).reshape(np_rows, 1)
    dinv, g0 = _tc_scale(degsum, _tc_matmul(x, W0, n, blk), n, blk)
    a0 = _sc_aggregate(g0, edge3, np_rows, base, rem, W0.shape[1])
    g1 = _tc_mid(a0, g0, dinv, b0, W1, n, blk)
    a1 = _sc_aggregate(g1, edge3, np_rows, base, rem, W1.shape[1])
    W2p = jnp.pad(W2, ((0, 0), (0, 128 - W2.shape[1])))
    g2 = _tc_mid(a1, g1, dinv, b1, W2p, n, blk)
    a2 = _sc_aggregate(g2, edge3, np_rows, base, rem, W2p.shape[1])
    return _tc_final(a2, g2, dinv, b2, n, blk)


# speedup vs baseline: 1.2248x; 1.0251x over previous
import functools

import jax
import jax.numpy as jnp
from jax import lax
from jax.experimental import pallas as pl
from jax.experimental.pallas import tpu as pltpu
from jax.experimental.pallas import tpu_sc as plsc

NC = 2
NS = 16
NW = NC * NS
C = 128
GRP = 32


def _fill(ref, n, value):
    def body(i, _):
        ref[pl.ds(i * 16, 16)] = jnp.full((16,), value, jnp.float32)
        return 0
    lax.fori_loop(0, n // 16, body, 0)


def _fill2d(ref, rows, cols, value):
    def body(i, _):
        r = i // (cols // 16)
        c = i % (cols // 16)
        ref[r, pl.ds(c * 16, 16)] = jnp.full((16,), value, jnp.float32)
        return 0
    lax.fori_loop(0, rows * (cols // 16), body, 0)


def _sc_degree(edge3, np_rows, base, rem):
    rows_per_tile = np_rows // NS
    full_groups, tail = base // GRP, base % GRP
    mesh = plsc.VectorSubcoreMesh(core_axis_name="c", subcore_axis_name="s")

    @functools.partial(
        pl.kernel,
        out_type=jax.ShapeDtypeStruct((NC, np_rows), jnp.float32),
        mesh=mesh,
        scratch_types=[
            pltpu.VMEM_SHARED((np_rows,), jnp.float32),
            pltpu.VMEM((GRP, 2, C), jnp.int32),
            pltpu.VMEM((C,), jnp.float32),
            pltpu.VMEM((rows_per_tile,), jnp.float32),
        ],
    )
    def deg_kernel(edge_hbm, out_hbm, acc, idx_v, ones_v, zeros_v):
        cid = lax.axis_index("c")
        sid = lax.axis_index("s")
        wid = cid * NS + sid
        start = base * wid + jnp.minimum(wid, rem)
        _fill(ones_v, C, 1.0)
        _fill(zeros_v, rows_per_tile, 0.0)
        pltpu.sync_copy(zeros_v, acc.at[pl.ds(sid * rows_per_tile, rows_per_tile)])
        plsc.subcore_barrier()

        def run(nchunks):
            def chunk(j, _):
                pltpu.sync_copy(ones_v, acc.at[idx_v.at[j, 1]], add=True)
                return 0
            lax.fori_loop(0, nchunks, chunk, 0)

        def group(gi, _):
            pltpu.sync_copy(edge_hbm.at[pl.ds(start + gi * GRP, GRP)], idx_v)
            run(GRP)
            return 0
        lax.fori_loop(0, full_groups, group, 0)
        if tail:
            pltpu.sync_copy(
                edge_hbm.at[pl.ds(start + full_groups * GRP, tail)],
                idx_v.at[pl.ds(0, tail)])
            run(tail)
        if rem:
            @pl.when(wid < rem)
            def _():
                pltpu.sync_copy(edge_hbm.at[pl.ds(start + base, 1)],
                                idx_v.at[pl.ds(0, 1)])
                pltpu.sync_copy(ones_v, acc.at[idx_v.at[0, 1]], add=True)
        plsc.subcore_barrier()
        pltpu.sync_copy(acc.at[pl.ds(sid * rows_per_tile, rows_per_tile)],
                        out_hbm.at[cid, pl.ds(sid * rows_per_tile, rows_per_tile)])

    return deg_kernel(edge3)


def _sc_aggregate(g, edge3, np_rows, base, rem, d):
    rows_per_tile = np_rows // NS
    full_groups, tail = base // GRP, base % GRP
    mesh = plsc.VectorSubcoreMesh(core_axis_name="c", subcore_axis_name="s")

    @functools.partial(
        pl.kernel,
        out_type=jax.ShapeDtypeStruct((NC, np_rows, d), jnp.float32),
        mesh=mesh,
        scratch_types=[
            pltpu.VMEM_SHARED((np_rows, d), jnp.float32),
            pltpu.VMEM((2, GRP, 2, C), jnp.int32),
            pltpu.VMEM((2, C, d), jnp.float32),
            pltpu.SemaphoreType.DMA,
            pltpu.SemaphoreType.DMA,
            pltpu.SemaphoreType.DMA,
        ],
    )
    def agg_kernel(g_hbm, edge_hbm, out_hbm, acc, idx_v, rows_v,
                   sem0, sem1, sem_i):
        cid = lax.axis_index("c")
        sid = lax.axis_index("s")
        wid = cid * NS + sid
        start = base * wid + jnp.minimum(wid, rem)

        _fill2d(rows_v.at[0], C, d, 0.0)
        zfull = rows_per_tile // C
        ztail = rows_per_tile - zfull * C
        for k in range(zfull):
            pltpu.async_copy(rows_v.at[0],
                             acc.at[pl.ds(sid * rows_per_tile + k * C, C)],
                             sem_i)
        if ztail:
            pltpu.async_copy(
                rows_v.at[0].at[pl.ds(0, ztail)],
                acc.at[pl.ds(sid * rows_per_tile + zfull * C, ztail)], sem_i)
        for k in range(zfull):
            pltpu.make_async_copy(
                rows_v.at[0],
                acc.at[pl.ds(sid * rows_per_tile + k * C, C)],
                sem_i).wait()
        if ztail:
            pltpu.make_async_copy(
                rows_v.at[0].at[pl.ds(0, ztail)],
                acc.at[pl.ds(sid * rows_per_tile + zfull * C, ztail)],
                sem_i).wait()
        plsc.subcore_barrier()

        def wait_gather(buf, sem):
            pltpu.make_async_copy(g_hbm.at[pl.ds(0, C)], buf, sem).wait()

        def pairs(iv, npairs):
            pltpu.async_copy(g_hbm.at[iv.at[0, 0]], rows_v.at[0], sem0)

            def pair(t, _):
                pltpu.async_copy(g_hbm.at[iv.at[2 * t + 1, 0]], rows_v.at[1],
                                 sem1)
                wait_gather(rows_v.at[0], sem0)
                pltpu.sync_copy(rows_v.at[0], acc.at[iv.at[2 * t, 1]],
                                add=True)

                @pl.when(t + 1 < npairs)
                def _():
                    pltpu.async_copy(g_hbm.at[iv.at[2 * t + 2, 0]],
                                     rows_v.at[0], sem0)
                wait_gather(rows_v.at[1], sem1)
                pltpu.sync_copy(rows_v.at[1], acc.at[iv.at[2 * t + 1, 1]],
                                add=True)
                return 0
            lax.fori_loop(0, npairs, pair, 0)

        pltpu.sync_copy(edge_hbm.at[pl.ds(start, GRP)], idx_v.at[0])

        def group(gi, _):
            s = gi % 2
            iv = idx_v.at[s]

            @pl.when(gi > 0)
            def _():
                pltpu.make_async_copy(edge_hbm.at[pl.ds(0, GRP)], iv,
                                      sem_i).wait()

            @pl.when(gi + 1 < full_groups)
            def _():
                pltpu.async_copy(
                    edge_hbm.at[pl.ds(start + (gi + 1) * GRP, GRP)],
                    idx_v.at[1 - s], sem_i)
            if tail:
                @pl.when(gi + 1 == full_groups)
                def _():
                    pltpu.async_copy(
                        edge_hbm.at[pl.ds(start + full_groups * GRP, tail)],
                        idx_v.at[1 - s].at[pl.ds(0, tail)], sem_i)
            pairs(iv, GRP // 2)
            return 0
        lax.fori_loop(0, full_groups, group, 0)

        if tail:
            s = full_groups % 2
            iv = idx_v.at[s]
            if full_groups:
                pltpu.make_async_copy(edge_hbm.at[pl.ds(0, tail)],
                                      iv.at[pl.ds(0, tail)], sem_i).wait()
            else:
                pltpu.sync_copy(edge_hbm.at[pl.ds(start, tail)],
                                iv.at[pl.ds(0, tail)])
            pairs(iv, tail // 2)
            if tail % 2:
                pltpu.async_copy(g_hbm.at[iv.at[tail - 1, 0]], rows_v.at[0],
                                 sem0)
                wait_gather(rows_v.at[0], sem0)
                pltpu.sync_copy(rows_v.at[0], acc.at[iv.at[tail - 1, 1]],
                                add=True)
        if rem:
            @pl.when(wid < rem)
            def _():
                pltpu.sync_copy(edge_hbm.at[pl.ds(start + base, 1)],
                                idx_v.at[0].at[pl.ds(0, 1)])
                pltpu.async_copy(g_hbm.at[idx_v.at[0, 0, 0]], rows_v.at[0],
                                 sem0)
                wait_gather(rows_v.at[0], sem0)
                pltpu.sync_copy(rows_v.at[0], acc.at[idx_v.at[0, 0, 1]],
                                add=True)
        plsc.subcore_barrier()
        pltpu.sync_copy(acc.at[pl.ds(sid * rows_per_tile, rows_per_tile)],
                        out_hbm.at[cid, pl.ds(sid * rows_per_tile, rows_per_tile)])

    return agg_kernel(g, edge3)


def _tc_matmul(x, w0, n, blk):
    din, dh = w0.shape

    def body(x_ref, w_ref, h_ref):
        h_ref[...] = jnp.dot(x_ref[...], w_ref[...],
                             preferred_element_type=jnp.float32)

    grid = (n // blk,)
    return pl.pallas_call(
        body,
        grid=grid,
        in_specs=[
            pl.BlockSpec((blk, din), lambda i: (i, 0)),
            pl.BlockSpec((din, dh), lambda i: (0, 0)),
        ],
        out_specs=pl.BlockSpec((blk, dh), lambda i: (i, 0)),
        out_shape=jax.ShapeDtypeStruct((n, dh), jnp.float32),
    )(x, w0)


def _tc_scale(degsum, h0, n, blk):
    dh = h0.shape[1]

    def body(deg_ref, h_ref, dinv_ref, g_ref):
        dv = lax.rsqrt(deg_ref[...] + 1.0)
        dinv_ref[...] = dv
        g_ref[...] = h_ref[...] * dv

    grid = (n // blk,)
    return pl.pallas_call(
        body,
        grid=grid,
        in_specs=[
            pl.BlockSpec((blk, 1), lambda i: (i, 0)),
            pl.BlockSpec((blk, dh), lambda i: (i, 0)),
        ],
        out_specs=[
            pl.BlockSpec((blk, 1), lambda i: (i, 0)),
            pl.BlockSpec((blk, dh), lambda i: (i, 0)),
        ],
        out_shape=[
            jax.ShapeDtypeStruct((n, 1), jnp.float32),
            jax.ShapeDtypeStruct((n, dh), jnp.float32),
        ],
    )(degsum, h0)


def _tc_mid(aggp, g, dinv, b, w, n, blk):
    d, dn = w.shape

    def body(aggp_ref, g_ref, dinv_ref, b_ref, w_ref, out_ref):
        s = aggp_ref[0] + aggp_ref[1] + g_ref[...]
        dv = dinv_ref[...]
        h = jnp.maximum(s * dv + b_ref[...][None, :], 0.0)
        out_ref[...] = jnp.dot(h, w_ref[...],
                               preferred_element_type=jnp.float32) * dv

    grid = (n // blk,)
    return pl.pallas_call(
        body,
        grid=grid,
        in_specs=[
            pl.BlockSpec((NC, blk, d), lambda i: (0, i, 0)),
            pl.BlockSpec((blk, d), lambda i: (i, 0)),
            pl.BlockSpec((blk, 1), lambda i: (i, 0)),
            pl.BlockSpec((d,), lambda i: (0,)),
            pl.BlockSpec((d, dn), lambda i: (0, 0)),
        ],
        out_specs=pl.BlockSpec((blk, dn), lambda i: (i, 0)),
        out_shape=jax.ShapeDtypeStruct((n, dn), jnp.float32),
    )(aggp, g, dinv, b, w)


def _tc_final(aggp, g, dinv, b, n, blk):
    d = g.shape[1]
    dout = b.shape[0]

    def body(aggp_ref, g_ref, dinv_ref, b_ref, out_ref):
        s = aggp_ref[0] + aggp_ref[1] + g_ref[...]
        v = (s * dinv_ref[...])[:, :dout] + b_ref[...][None, :]
        m = jnp.max(v, axis=-1, keepdims=True)
        e = v - m
        out_ref[...] = e - jnp.log(jnp.sum(jnp.exp(e), axis=-1, keepdims=True))

    grid = (n // blk,)
    return pl.pallas_call(
        body,
        grid=grid,
        in_specs=[
            pl.BlockSpec((NC, blk, d), lambda i: (0, i, 0)),
            pl.BlockSpec((blk, d), lambda i: (i, 0)),
            pl.BlockSpec((blk, 1), lambda i: (i, 0)),
            pl.BlockSpec((dout,), lambda i: (0,)),
        ],
        out_specs=pl.BlockSpec((blk, dout), lambda i: (i, 0)),
        out_shape=jax.ShapeDtypeStruct((n, dout), jnp.float32),
    )(aggp, g, dinv, b)


def kernel(x, edge_index, W0, b0, W1, b1, W2, b2):
    n, din = x.shape
    e = edge_index.shape[1]

    np_deg = ((n + 128 * NS - 1) // (128 * NS)) * (128 * NS)
    np_rows = ((n + 8 * NS - 1) // (8 * NS)) * (8 * NS)
    n_trash = np_rows - n

    e_chunks = e // C
    base, rem = e_chunks // NW, e_chunks % NW
    edge3 = edge_index.reshape(2, e_chunks, C).transpose(1, 0, 2)

    blk = 5000
    degp = _sc_degree(edge3, np_deg, base, rem)
    degsum = (degp[0] + degp[1]).reshape(np_deg, 1)
    dinv, g0 = _tc_scale(degsum, _tc_matmul(x, W0, n, blk), n, blk)
    a0 = _sc_aggregate(g0, edge3, np_rows, base, rem, W0.shape[1])
    g1 = _tc_mid(a0, g0, dinv, b0, W1, n, blk)
    a1 = _sc_aggregate(g1, edge3, np_rows, base, rem, W1.shape[1])
    W2p = jnp.pad(W2, ((0, 0), (0, 128 - W2.shape[1])))
    g2 = _tc_mid(a1, g1, dinv, b1, W2p, n, blk)
    a2 = _sc_aggregate(g2, edge3, np_rows, base, rem, W2p.shape[1])
    return _tc_final(a2, g2, dinv, b2, n, blk)
